# baseline (speedup 1.0000x reference)
import jax
import jax.numpy as jnp
from jax.experimental import pallas as pl
from jax.experimental.pallas import tpu as pltpu

_NBUF = 4
_RT = 256
_H = _RT // 2


def _make_body(B, D, S, n_chunks, chunks_per_b):
    def body(labels_ref, x_ref, table_ref, o_ref,
             xbuf, obuf, ebuf, in_sems, out_sems, emb_sem):
        emb_copies = [
            pltpu.make_async_copy(
                table_ref.at[labels_ref[b]], ebuf.at[b], emb_sem)
            for b in range(B)
        ]
        for cp in emb_copies:
            cp.start()

        def in_copies(c, slot):
            base = c * _RT
            return [
                pltpu.make_async_copy(
                    x_ref.at[pl.ds(base + h * _H, _H)],
                    xbuf.at[slot, pl.ds(h * _H, _H)],
                    in_sems.at[slot, h])
                for h in range(2)
            ]

        def out_copies(c, slot):
            base = c * _RT
            return [
                pltpu.make_async_copy(
                    obuf.at[slot, pl.ds(h * _H, _H)],
                    o_ref.at[pl.ds(base + h * _H, _H)],
                    out_sems.at[slot, h])
                for h in range(2)
            ]

        for i in range(_NBUF):
            for cp in in_copies(i, i):
                cp.start()
        for cp in emb_copies:
            cp.wait()

        def step(c, carry):
            slot = jax.lax.rem(c, _NBUF)
            for cp in in_copies(c, slot):
                cp.wait()

            @pl.when(c >= _NBUF)
            def _():
                for cp in out_copies(c - _NBUF, slot):
                    cp.wait()

            b = c // chunks_per_b
            j = jax.lax.rem(c, chunks_per_b)
            e = ebuf[b, j, :]
            obuf[slot] = xbuf[slot] + e[:, None]
            for cp in out_copies(c, slot):
                cp.start()

            @pl.when(c + _NBUF < n_chunks)
            def _():
                for cp in in_copies(c + _NBUF, slot):
                    cp.start()

            return carry

        jax.lax.fori_loop(0, n_chunks, step, 0)
        for k in range(_NBUF):
            c = n_chunks - _NBUF + k
            for cp in out_copies(c, jax.lax.rem(jnp.int32(c), _NBUF)):
                cp.wait()

    return body


def kernel(x, spec_labels, table):
    B, D, S = x.shape
    n_rows = B * D
    n_chunks = n_rows // _RT
    chunks_per_b = D // _RT
    x2 = x.reshape(n_rows, S)
    table3 = table.reshape(table.shape[0], chunks_per_b, _RT)

    out = pl.pallas_call(
        _make_body(B, D, S, n_chunks, chunks_per_b),
        in_specs=[
            pl.BlockSpec(memory_space=pltpu.SMEM),
            pl.BlockSpec(memory_space=pl.ANY),
            pl.BlockSpec(memory_space=pl.ANY),
        ],
        out_specs=pl.BlockSpec(memory_space=pl.ANY),
        out_shape=jax.ShapeDtypeStruct((n_rows, S), x.dtype),
        scratch_shapes=[
            pltpu.VMEM((_NBUF, _RT, S), x.dtype),
            pltpu.VMEM((_NBUF, _RT, S), x.dtype),
            pltpu.VMEM((B, chunks_per_b, _RT), x.dtype),
            pltpu.SemaphoreType.DMA((_NBUF, 2)),
            pltpu.SemaphoreType.DMA((_NBUF, 2)),
            pltpu.SemaphoreType.DMA,
        ],
        compiler_params=pltpu.CompilerParams(
            vmem_limit_bytes=64 * 1024 * 1024,
        ),
    )(spec_labels.astype(jnp.int32), x2, table3)
    return out.reshape(B, D, S)


# speedup vs baseline: 1.2505x; 1.2505x over previous
import jax
import jax.numpy as jnp
from jax.experimental import pallas as pl
from jax.experimental.pallas import tpu as pltpu


def _spec_add_kernel(labels_ref, x_ref, emb_ref, o_ref):
    e = emb_ref[0, 0, :]
    o_ref[...] = x_ref[...] + e[None, :, None]


def kernel(x, spec_labels, table):
    B, D, S = x.shape
    Dt = 512
    grid = (B, D // Dt)
    table3 = table.reshape(table.shape[0], 1, D)
    grid_spec = pltpu.PrefetchScalarGridSpec(
        num_scalar_prefetch=1,
        grid=grid,
        in_specs=[
            pl.BlockSpec((1, Dt, S), lambda b, d, labels: (b, d, 0)),
            pl.BlockSpec((1, 1, Dt), lambda b, d, labels: (labels[b], 0, d)),
        ],
        out_specs=pl.BlockSpec((1, Dt, S), lambda b, d, labels: (b, d, 0)),
    )
    return pl.pallas_call(
        _spec_add_kernel,
        grid_spec=grid_spec,
        out_shape=jax.ShapeDtypeStruct((B, D, S), x.dtype),
        compiler_params=pltpu.CompilerParams(
            dimension_semantics=("parallel", "parallel"),
            vmem_limit_bytes=64 * 1024 * 1024,
        ),
    )(spec_labels.astype(jnp.int32), x, table3)
